# baseline (speedup 1.0000x reference)
import functools

import jax
import jax.numpy as jnp
from jax import lax
from jax.experimental import pallas as pl
from jax.experimental.pallas import tpu as pltpu
from jax.experimental.pallas import tpu_sc as plsc

_L = 16


@functools.cache
def _make_tableT(V, D):
    info = plsc.get_sparse_core_info()
    NC, NS = info.num_cores, info.num_subcores
    NW = NC * NS
    TCOLS = V // 128
    TAIL = (V - TCOLS * 128) * D
    n_iter = (TCOLS + 2 * NW - 1) // (2 * NW)
    mesh = plsc.VectorSubcoreMesh(core_axis_name="c", subcore_axis_name="s")

    @functools.partial(
        pl.kernel,
        mesh=mesh,
        out_type=jax.ShapeDtypeStruct((V * D,), jnp.float32),
        scratch_types=[
            pltpu.VMEM((2, D, 129), jnp.float32),
            pltpu.VMEM((2, 128 * D), jnp.float32),
            pltpu.VMEM((TAIL,), jnp.float32),
            pltpu.SemaphoreType.DMA((2,)),
            pltpu.SemaphoreType.DMA((2,)),
        ],
        compiler_params=pltpu.CompilerParams(
            use_tc_tiling_on_sc=True, needs_layout_passes=False),
    )
    def k(tt_hbm, tail_hbm, out_hbm, st_v, tr_v, tail_v, lsem, ssem):
        wid = lax.axis_index("s") * NC + lax.axis_index("c")
        lanes = lax.iota(jnp.int32, _L)

        @pl.when(wid == 0)
        def _():
            pltpu.sync_copy(tail_hbm, tail_v)
            pltpu.sync_copy(tail_v, out_hbm.at[pl.ds(TCOLS * 128 * D, TAIL)])

        def load_cp(c, b):
            return pltpu.make_async_copy(
                tt_hbm.at[:, pl.ds(c * 128, 128)],
                st_v.at[b].at[:, pl.ds(0, 128)], lsem.at[b])

        def store_cp(c, b):
            return pltpu.make_async_copy(
                tr_v.at[b], out_hbm.at[pl.ds(c * 128 * D, 128 * D)],
                ssem.at[b])

        def ld(c, b):
            load_cp(c, b).start()

        def wait_ld(c, b):
            load_cp(c, b).wait()

        def st(c, b):
            store_cp(c, b).start()

        def wait_st(c, b):
            store_cp(c, b).wait()

        def transpose_block(b):
            def i_body(i, carry):
                i_vec = jnp.full((_L,), i, jnp.int32)
                for hf in range(D // _L):
                    vec = plsc.load_gather(
                        st_v.at[b], [hf * _L + lanes, i_vec])
                    tr_v[b, pl.ds(i * D + hf * _L, _L)] = vec
                return carry
            lax.fori_loop(0, 128, i_body, 0, unroll=8)

        ld(wid, 0)

        def half(t, b):
            k = 2 * t + b
            c = wid + NW * k

            @pl.when(c < TCOLS)
            def _():
                wait_ld(c, b)
                cn = c + NW
                @pl.when(cn < TCOLS)
                def _():
                    ld(cn, 1 - b)
                @pl.when(k >= 2)
                def _():
                    wait_st(c - 2 * NW, b)
                transpose_block(b)
                st(c, b)

        def body(t, carry):
            half(t, 0)
            half(t, 1)
            return carry

        lax.fori_loop(0, n_iter, body, 0)

        k_last = (TCOLS - 1 - wid) // NW
        p = k_last % 2
        c_last = wid + NW * k_last

        def wait_st_dyn(c, b):
            pltpu.make_async_copy(
                tr_v.at[b], out_hbm.at[pl.ds(c * 128 * D, 128 * D)],
                ssem.at[b]).wait()

        wait_st_dyn(c_last, p)
        wait_st_dyn(c_last - NW, 1 - p)

    return k


@functools.cache
def _make_gather(BT, H, D):
    info = plsc.get_sparse_core_info()
    NC, NS = info.num_cores, info.num_subcores
    NW = NC * NS
    assert BT % (NW * 128) == 0 and D % 8 == 0 and H % 2 == 0
    W = BT // NW
    E1, B1 = D // 8, BT // 128
    WB = W // 128
    mesh = plsc.VectorSubcoreMesh(core_axis_name="c", subcore_axis_name="s")

    @functools.partial(
        pl.kernel,
        mesh=mesh,
        out_type=jax.ShapeDtypeStruct((H, E1, B1, 8, 128), jnp.float32),
        scratch_types=[
            pltpu.VMEM((H, W), jnp.int32),
            pltpu.VMEM((2, W, D), jnp.float32),
            pltpu.VMEM((2, E1, WB, 8, 129), jnp.float32),
            pltpu.SemaphoreType.DMA((2,)),
            pltpu.SemaphoreType.DMA((2,)),
        ],
        compiler_params=pltpu.CompilerParams(
            use_tc_tiling_on_sc=False, needs_layout_passes=False),
    )
    def k(idx_hbm, table_hbm, y_hbm, idxT_v, rows_v, rowsT_v, gsem, ssem):
        wid = lax.axis_index("s") * NC + lax.axis_index("c")
        pltpu.sync_copy(idx_hbm.at[:, pl.ds(wid * W, W)], idxT_v)
        lanes = lax.iota(jnp.int32, _L)

        def gather_cp(h, b):
            return pltpu.make_async_copy(
                table_hbm.at[idxT_v.at[h]], rows_v.at[b], gsem.at[b])

        def store_cp(h, b):
            return pltpu.make_async_copy(
                rowsT_v.at[b].at[:, :, :, pl.ds(0, 128)],
                y_hbm.at[h, :, pl.ds(wid * WB, WB)], ssem.at[b])

        half_idx = [((2 * hf + lanes // 8), (lanes % 8)) for hf in range(D // _L)]

        def transpose_rows(b):
            def c_body(c, carry):
                c_vec = jnp.full((_L,), c, jnp.int32)

                def b_body(b0, carry2):
                    b_vec = jnp.full((_L,), b0, jnp.int32)
                    row = c * 128 + b0
                    for hf, (r_vec, e_vec) in enumerate(half_idx):
                        vec = rows_v[b, row, pl.ds(hf * _L, _L)]
                        plsc.store_scatter(
                            rowsT_v.at[b], [r_vec, c_vec, e_vec, b_vec], vec)
                    return carry2
                lax.fori_loop(0, 128, b_body, 0, unroll=8)
                return carry
            lax.fori_loop(0, WB, c_body, 0)

        gather_cp(0, 0).start()

        def half(t, b):
            h = 2 * t + b
            gather_cp(h, b).wait()
            if b == 0:
                gather_cp(h + 1, 1 - b).start()
            else:
                @pl.when(t < H // 2 - 1)
                def _():
                    gather_cp(h + 1, 1 - b).start()

            @pl.when(t > 0)
            def _():
                store_cp(h - 2, b).wait()
            transpose_rows(b)
            store_cp(h, b).start()

        def body(t, carry):
            half(t, 0)
            half(t, 1)
            return carry

        lax.fori_loop(0, H // 2, body, 0)
        store_cp(H - 2, 0).wait()
        store_cp(H - 1, 1).wait()

    return k


def kernel(input_tensor, table):
    bt, h = input_tensor.shape
    v, d = table.shape
    tail = table[(v // 128) * 128:].reshape(-1)
    table_rm = _make_tableT(v, d)(table.T, tail).reshape(v, d)
    y = _make_gather(bt, h, d)(input_tensor.T, table_rm)
    return y.transpose(2, 4, 0, 1, 3).reshape(bt, h, d)


# speedup vs baseline: 1.4304x; 1.4304x over previous
import functools

import jax
import jax.numpy as jnp
from jax import lax
from jax.experimental import pallas as pl
from jax.experimental.pallas import tpu as pltpu
from jax.experimental.pallas import tpu_sc as plsc

_L = 16


@functools.cache
def _make_tableT(V, D):
    info = plsc.get_sparse_core_info()
    NC, NS = info.num_cores, info.num_subcores
    NW = NC * NS
    TCOLS = V // 128
    TAIL = (V - TCOLS * 128) * D
    n_iter = (TCOLS + 2 * NW - 1) // (2 * NW)
    mesh = plsc.VectorSubcoreMesh(core_axis_name="c", subcore_axis_name="s")

    @functools.partial(
        pl.kernel,
        mesh=mesh,
        out_type=jax.ShapeDtypeStruct((V * D,), jnp.float32),
        scratch_types=[
            pltpu.VMEM((2, D // 8, 8, 128), jnp.float32),
            pltpu.VMEM((2, 128 * D), jnp.float32),
            pltpu.VMEM((128 * 33 + D, ), jnp.float32),
            pltpu.VMEM((TAIL,), jnp.float32),
            pltpu.SemaphoreType.DMA((2,)),
            pltpu.SemaphoreType.DMA((2,)),
        ],
        compiler_params=pltpu.CompilerParams(
            use_tc_tiling_on_sc=True, needs_layout_passes=False),
    )
    def k(tt_hbm, tail_hbm, out_hbm, st_v, tr_v, tmp_v, tail_v, lsem, ssem):
        wid = lax.axis_index("s") * NC + lax.axis_index("c")
        lanes = lax.iota(jnp.int32, _L)

        @pl.when(wid == 0)
        def _():
            pltpu.sync_copy(tail_hbm, tail_v)
            pltpu.sync_copy(tail_v, out_hbm.at[pl.ds(TCOLS * 128 * D, TAIL)])

        def load_cps(c, b):
            return [pltpu.make_async_copy(
                tt_hbm.at[pl.ds(r * 8, 8), pl.ds(c * 128, 128)],
                st_v.at[b, r], lsem.at[b]) for r in range(D // 8)]

        def store_cp(c, b):
            return pltpu.make_async_copy(
                tr_v.at[b], out_hbm.at[pl.ds(c * 128 * D, 128 * D)],
                ssem.at[b])

        def ld(c, b):
            for cp in load_cps(c, b):
                cp.start()

        def wait_ld(c, b):
            for cp in load_cps(c, b):
                cp.wait()

        def st(c, b):
            store_cp(c, b).start()

        def wait_st(c, b):
            store_cp(c, b).wait()

        def transpose_block(b):
            def p1_body(t, carry):
                r = t // 64
                e0 = (t // 8) % 8
                i16 = (t % 8) * _L
                e = r * 8 + e0
                vec = st_v[b, r, e0, pl.ds(i16, _L)]
                addr = (i16 + lanes) * 33 + e
                plsc.store_scatter(tmp_v, [addr], vec)
                return carry
            lax.fori_loop(0, (D // 8) * 8 * 8, p1_body, 0, unroll=8)

            def p2_body(t, carry):
                i = t // (D // _L)
                hf = t % (D // _L)
                vec = tmp_v[pl.ds(i * 33 + hf * _L, _L)]
                tr_v[b, pl.ds(i * D + hf * _L, _L)] = vec
                return carry
            lax.fori_loop(0, 128 * (D // _L), p2_body, 0, unroll=8)

        ld(wid, 0)

        def half(t, b):
            k = 2 * t + b
            c = wid + NW * k

            @pl.when(c < TCOLS)
            def _():
                wait_ld(c, b)
                cn = c + NW
                @pl.when(cn < TCOLS)
                def _():
                    ld(cn, 1 - b)
                @pl.when(k >= 2)
                def _():
                    wait_st(c - 2 * NW, b)
                transpose_block(b)
                st(c, b)

        def body(t, carry):
            half(t, 0)
            half(t, 1)
            return carry

        lax.fori_loop(0, n_iter, body, 0)

        k_last = (TCOLS - 1 - wid) // NW
        p = k_last % 2
        c_last = wid + NW * k_last

        def wait_st_dyn(c, b):
            pltpu.make_async_copy(
                tr_v.at[b], out_hbm.at[pl.ds(c * 128 * D, 128 * D)],
                ssem.at[b]).wait()

        wait_st_dyn(c_last, p)
        wait_st_dyn(c_last - NW, 1 - p)

    return k


@functools.cache
def _make_gather(BT, H, D):
    info = plsc.get_sparse_core_info()
    NC, NS = info.num_cores, info.num_subcores
    NW = NC * NS
    assert BT % (NW * 128) == 0 and D % 8 == 0 and H % 2 == 0
    W = BT // NW
    E1, B1 = D // 8, BT // 128
    WB = W // 128
    mesh = plsc.VectorSubcoreMesh(core_axis_name="c", subcore_axis_name="s")

    @functools.partial(
        pl.kernel,
        mesh=mesh,
        out_type=jax.ShapeDtypeStruct((H, E1, B1, 8, 128), jnp.float32),
        scratch_types=[
            pltpu.VMEM((H, W), jnp.int32),
            pltpu.VMEM((2, W, D), jnp.float32),
            pltpu.VMEM((2, E1, WB, 8, 129), jnp.float32),
            pltpu.SemaphoreType.DMA((2,)),
            pltpu.SemaphoreType.DMA((2,)),
        ],
        compiler_params=pltpu.CompilerParams(
            use_tc_tiling_on_sc=False, needs_layout_passes=False),
    )
    def k(idx_hbm, table_hbm, y_hbm, idxT_v, rows_v, rowsT_v, gsem, ssem):
        wid = lax.axis_index("s") * NC + lax.axis_index("c")
        pltpu.sync_copy(idx_hbm.at[:, pl.ds(wid * W, W)], idxT_v)
        lanes = lax.iota(jnp.int32, _L)

        def gather_cp(h, b):
            return pltpu.make_async_copy(
                table_hbm.at[idxT_v.at[h]], rows_v.at[b], gsem.at[b])

        def store_cp(h, b):
            return pltpu.make_async_copy(
                rowsT_v.at[b].at[:, :, :, pl.ds(0, 128)],
                y_hbm.at[h, :, pl.ds(wid * WB, WB)], ssem.at[b])

        half_idx = [((2 * hf + lanes // 8), (lanes % 8)) for hf in range(D // _L)]

        def transpose_rows(b):
            def c_body(c, carry):
                c_vec = jnp.full((_L,), c, jnp.int32)

                def b_body(b0, carry2):
                    b_vec = jnp.full((_L,), b0, jnp.int32)
                    row = c * 128 + b0
                    for hf, (r_vec, e_vec) in enumerate(half_idx):
                        vec = rows_v[b, row, pl.ds(hf * _L, _L)]
                        plsc.store_scatter(
                            rowsT_v.at[b], [r_vec, c_vec, e_vec, b_vec], vec)
                    return carry2
                lax.fori_loop(0, 128, b_body, 0, unroll=8)
                return carry
            lax.fori_loop(0, WB, c_body, 0)

        gather_cp(0, 0).start()

        def half(t, b):
            h = 2 * t + b
            gather_cp(h, b).wait()
            if b == 0:
                gather_cp(h + 1, 1 - b).start()
            else:
                @pl.when(t < H // 2 - 1)
                def _():
                    gather_cp(h + 1, 1 - b).start()

            @pl.when(t > 0)
            def _():
                store_cp(h - 2, b).wait()
            transpose_rows(b)
            store_cp(h, b).start()

        def body(t, carry):
            half(t, 0)
            half(t, 1)
            return carry

        lax.fori_loop(0, H // 2, body, 0)
        store_cp(H - 2, 0).wait()
        store_cp(H - 1, 1).wait()

    return k


def kernel(input_tensor, table):
    bt, h = input_tensor.shape
    v, d = table.shape
    tail = table[(v // 128) * 128:].reshape(-1)
    table_rm = _make_tableT(v, d)(table.T, tail).reshape(v, d)
    y = _make_gather(bt, h, d)(input_tensor.T, table_rm)
    return y.transpose(2, 4, 0, 1, 3).reshape(bt, h, d)
